# baseline (speedup 1.0000x reference)
import functools

import jax
import jax.numpy as jnp
from jax import lax
from jax.experimental import pallas as pl
from jax.experimental.pallas import tpu as pltpu
from jax.experimental.pallas import tpu_sc as plsc

NC = 2
NS = 16
NW = NC * NS
L = 16

H = 128
CHE = 64
ECH = 128


def _emb_body(x0_hbm, x1_hbm, kt_hbm, vt_hbm, out_hbm,
              i0_v, i1_v, ka_v, vb_v, sem0, sem1, *, rpt):
    c = lax.axis_index("c")
    s = lax.axis_index("s")
    wid = s * NC + c
    nstep = rpt // CHE
    pltpu.sync_copy(x0_hbm.at[pl.ds(wid * rpt, rpt)], i0_v)
    pltpu.sync_copy(x1_hbm.at[pl.ds(wid * rpt, rpt)], i1_v)

    def step(t, carry):
        sl = pl.ds(t * CHE, CHE)
        pltpu.async_copy(kt_hbm.at[i0_v.at[sl]], ka_v, sem0)
        pltpu.async_copy(vt_hbm.at[i1_v.at[sl]], vb_v, sem1)
        pltpu.make_async_copy(kt_hbm.at[i0_v.at[sl]], ka_v, sem0).wait()
        pltpu.make_async_copy(vt_hbm.at[i1_v.at[sl]], vb_v, sem1).wait()

        def addrow(r, carry2):
            for cc in range(H // L):
                sl = pl.ds(cc * L, L)
                ka_v[r, sl] = ka_v[r, sl] + vb_v[r, sl]
            return carry2

        lax.fori_loop(0, CHE, addrow, 0)
        pltpu.sync_copy(ka_v, out_hbm.at[pl.ds(wid * rpt + t * CHE, CHE)])
        return carry

    lax.fori_loop(0, nstep, step, 0)


def _seg_body(h_hbm, pk_hbm, out_hbm,
              pk_v, sidx_v, didx_v, rows_v, hh_sh, agg_sh, sem0, sem1,
              *, np_, wb):
    c = lax.axis_index("c")
    s = lax.axis_index("s")
    wid = s * NC + c
    half_n = np_ // 2
    agg_rows = half_n + ECH
    rps_h = half_n // NS
    nzch = agg_rows // ECH

    zero = jnp.zeros((L,), jnp.float32)

    def zrow(r, carry):
        for cc in range(H // L):
            rows_v[0, r, pl.ds(cc * L, L)] = zero
        return carry

    def zero_agg():
        lax.fori_loop(0, ECH, zrow, 0)

        def zcopy(k, carry):
            q = s + k * NS

            @pl.when(q < nzch)
            def _():
                pltpu.sync_copy(rows_v.at[0],
                                agg_sh.at[pl.ds(q * ECH, ECH)])
            return carry

        lax.fori_loop(0, -(-nzch // NS), zcopy, 0)

    def stage_h(sh):
        plsc.subcore_barrier()
        pltpu.sync_copy(h_hbm.at[pl.ds(sh * half_n + s * rps_h, rps_h)],
                        hh_sh.at[pl.ds(s * rps_h, rps_h)])
        plsc.subcore_barrier()

    def flush(dh):
        plsc.subcore_barrier()
        pltpu.sync_copy(agg_sh.at[pl.ds(s * rps_h, rps_h)],
                        out_hbm.at[c, pl.ds(dh * half_n + s * rps_h, rps_h)])

    def gstart(j, b):
        if b == 0:
            pltpu.async_copy(hh_sh.at[sidx_v.at[j]], rows_v.at[0], sem0)
        else:
            pltpu.async_copy(hh_sh.at[sidx_v.at[j]], rows_v.at[1], sem1)

    def gwait(j, b):
        if b == 0:
            pltpu.make_async_copy(hh_sh.at[sidx_v.at[j]], rows_v.at[0], sem0).wait()
        else:
            pltpu.make_async_copy(hh_sh.at[sidx_v.at[j]], rows_v.at[1], sem1).wait()

    def scat(j, b):
        pltpu.sync_copy(rows_v.at[b], agg_sh.at[didx_v.at[j]], add=True)

    def phase(b):
        pltpu.sync_copy(pk_hbm.at[pl.ds((b * NW + wid) * wb, wb)], pk_v)

        def unpack(j, carry):
            for cc in range(H // L):
                sl = pl.ds(cc * L, L)
                v = pk_v[j, sl]
                sidx_v[j, sl] = lax.shift_right_logical(v, 13)
                didx_v[j, sl] = lax.bitwise_and(v, 8191)
            return carry

        lax.fori_loop(0, wb, unpack, 0)

        gstart(0, 0)

        def pair(j2, carry):
            j = 2 * j2
            gstart(j + 1, 1)
            gwait(j, 0)
            scat(j, 0)

            @pl.when(j2 < wb // 2 - 1)
            def _():
                gstart(j + 2, 0)

            gwait(j + 1, 1)
            scat(j + 1, 1)
            return carry

        lax.fori_loop(0, wb // 2, pair, 0)

    zero_agg()
    stage_h(0)
    phase(0)
    stage_h(1)
    phase(2)
    flush(0)
    plsc.subcore_barrier()
    zero_agg()
    plsc.subcore_barrier()
    phase(3)
    stage_h(0)
    phase(1)
    flush(1)


def _make_emb(np_):
    rpt = np_ // NW
    mesh = plsc.VectorSubcoreMesh(core_axis_name="c", subcore_axis_name="s",
                                  num_cores=NC, num_subcores=NS)
    return pl.kernel(
        functools.partial(_emb_body, rpt=rpt),
        out_type=jax.ShapeDtypeStruct((np_, H), jnp.float32),
        mesh=mesh,
        scratch_types=[
            pltpu.VMEM((rpt,), jnp.int32),
            pltpu.VMEM((rpt,), jnp.int32),
            pltpu.VMEM((CHE, H), jnp.float32),
            pltpu.VMEM((CHE, H), jnp.float32),
            pltpu.SemaphoreType.DMA,
            pltpu.SemaphoreType.DMA,
        ],
    )


def _make_seg(np_, wb):
    mesh = plsc.VectorSubcoreMesh(core_axis_name="c", subcore_axis_name="s",
                                  num_cores=NC, num_subcores=NS)
    return pl.kernel(
        functools.partial(_seg_body, np_=np_, wb=wb),
        out_type=jax.ShapeDtypeStruct((NC, np_, H), jnp.float32),
        mesh=mesh,
        scratch_types=[
            pltpu.VMEM((wb, ECH), jnp.int32),
            pltpu.VMEM((wb, ECH), jnp.int32),
            pltpu.VMEM((wb, ECH), jnp.int32),
            pltpu.VMEM((2, ECH, H), jnp.float32),
            pltpu.VMEM_SHARED((np_ // 2, H), jnp.float32),
            pltpu.VMEM_SHARED((np_ // 2 + ECH, H), jnp.float32),
            pltpu.SemaphoreType.DMA,
            pltpu.SemaphoreType.DMA,
        ],
    )


def _dense_mid_body(h_ref, p0_ref, p1_ref, ws_ref, wn_ref, bgb_ref, out_ref):
    h = h_ref[...]
    agg = p0_ref[...] + p1_ref[...]
    t = (jnp.dot(h, ws_ref[...], preferred_element_type=jnp.float32)
         + jnp.dot(agg, wn_ref[...], preferred_element_type=jnp.float32)
         + bgb_ref[0:1, :])
    hn = h + jnp.maximum(t, 0.0)
    mu = jnp.mean(hn, axis=-1, keepdims=True)
    var = jnp.mean((hn - mu) ** 2, axis=-1, keepdims=True)
    out_ref[...] = ((hn - mu) / jnp.sqrt(var + 1e-5) * bgb_ref[1:2, :]
                    + bgb_ref[2:3, :])


def _dense_last_body(h_ref, p0_ref, p1_ref, ws_ref, wn_ref, bgb_ref,
                     wout_ref, mask_ref, out_ref):
    h = h_ref[...]
    agg = p0_ref[...] + p1_ref[...]
    t = (jnp.dot(h, ws_ref[...], preferred_element_type=jnp.float32)
         + jnp.dot(agg, wn_ref[...], preferred_element_type=jnp.float32)
         + bgb_ref[0:1, :])
    hn = h + jnp.maximum(t, 0.0)
    mu = jnp.mean(hn, axis=-1, keepdims=True)
    var = jnp.mean((hn - mu) ** 2, axis=-1, keepdims=True)
    hln = ((hn - mu) / jnp.sqrt(var + 1e-5) * bgb_ref[1:2, :]
           + bgb_ref[2:3, :])
    out_ref[...] = (jnp.dot(hln, wout_ref[...], preferred_element_type=jnp.float32)
                    * mask_ref[...])


def _dense_mid(h, p0, p1, ws, wn, bgb, *, np_, blk=2048):
    grid = (np_ // blk,)
    return pl.pallas_call(
        _dense_mid_body,
        grid=grid,
        in_specs=[
            pl.BlockSpec((blk, H), lambda i: (i, 0)),
            pl.BlockSpec((blk, H), lambda i: (i, 0)),
            pl.BlockSpec((blk, H), lambda i: (i, 0)),
            pl.BlockSpec((H, H), lambda i: (0, 0)),
            pl.BlockSpec((H, H), lambda i: (0, 0)),
            pl.BlockSpec((3, H), lambda i: (0, 0)),
        ],
        out_specs=pl.BlockSpec((blk, H), lambda i: (i, 0)),
        out_shape=jax.ShapeDtypeStruct((np_, H), jnp.float32),
    )(h, p0, p1, ws, wn, bgb)


def _dense_last(h, p0, p1, ws, wn, bgb, wout_t, maskf, *, np_, blk=2048):
    grid = (np_ // blk,)
    return pl.pallas_call(
        _dense_last_body,
        grid=grid,
        in_specs=[
            pl.BlockSpec((blk, H), lambda i: (i, 0)),
            pl.BlockSpec((blk, H), lambda i: (i, 0)),
            pl.BlockSpec((blk, H), lambda i: (i, 0)),
            pl.BlockSpec((H, H), lambda i: (0, 0)),
            pl.BlockSpec((H, H), lambda i: (0, 0)),
            pl.BlockSpec((3, H), lambda i: (0, 0)),
            pl.BlockSpec((H, H), lambda i: (0, 0)),
            pl.BlockSpec((blk, 1), lambda i: (i, 0)),
        ],
        out_specs=pl.BlockSpec((blk, H), lambda i: (i, 0)),
        out_shape=jax.ShapeDtypeStruct((np_, H), jnp.float32),
    )(h, p0, p1, ws, wn, bgb, wout_t, maskf)


def kernel(x, edge_index, batch, root_mask, key_table, val_table,
           Ws, Wn, bias, ln_g, ln_b, W_out):
    n, _ = x.shape
    e = edge_index.shape[1]
    depth = Ws.shape[0]

    np_ = ((n + NW * CHE - 1) // (NW * CHE)) * (NW * CHE)
    half = np_ // 2
    wb = -(-(3 * e) // (10 * NW * ECH))
    wb += wb % 2
    cap = wb * NW * ECH

    pad_n = np_ - n
    x0 = jnp.concatenate([x[:, 0], jnp.zeros((pad_n,), jnp.int32)])
    x1 = jnp.concatenate([x[:, 1], jnp.zeros((pad_n,), jnp.int32)])

    s0, d0 = edge_index[0], edge_index[1]
    sl = (s0 >= half).astype(jnp.int32)
    dl = (d0 >= half).astype(jnp.int32)
    key = sl * 2 + dl
    packed = (s0 - sl * half) * 8192 + (d0 - dl * half)
    pos = jnp.zeros((e,), jnp.int32)
    for k in range(4):
        m = key == k
        r = jnp.cumsum(m.astype(jnp.int32)) - 1
        pos = jnp.where(m, k * cap + r, pos)
    j = jnp.arange(4 * cap, dtype=jnp.int32)
    pk_def = (j % jnp.int32(half)) * 8192 + jnp.int32(half) + (j % jnp.int32(ECH))
    pk = pk_def.at[pos].set(packed, unique_indices=True,
                            mode="promise_in_bounds")
    pk2d = pk.reshape(4 * cap // ECH, ECH)

    h = _make_emb(np_)(x0, x1, key_table, val_table)

    seg = _make_seg(np_, wb)
    maskf = jnp.concatenate([root_mask.astype(jnp.float32),
                             jnp.zeros((pad_n,), jnp.float32)]).reshape(np_, 1)
    wout_t = W_out.T

    preds = None
    for i in range(depth):
        parts = seg(h, pk2d)
        bgb = jnp.stack([bias[i], ln_g[i], ln_b[i]])
        if i < depth - 1:
            h = _dense_mid(h, parts[0], parts[1], Ws[i], Wn[i], bgb, np_=np_)
        else:
            preds = _dense_last(h, parts[0], parts[1], Ws[i], Wn[i], bgb,
                                wout_t, maskf, np_=np_)
    return preds[:n]


# speedup vs baseline: 1.5829x; 1.5829x over previous
import functools

import jax
import jax.numpy as jnp
from jax import lax
from jax.experimental import pallas as pl
from jax.experimental.pallas import tpu as pltpu
from jax.experimental.pallas import tpu_sc as plsc

NC = 2
NS = 16
NW = NC * NS
L = 16

H = 128
CHE = 64
ECH = 128


def _emb_body(x0_hbm, x1_hbm, kt_hbm, vt_hbm, out_hbm,
              i0_v, i1_v, ka_v, vb_v, sem0, sem1, *, rpt):
    c = lax.axis_index("c")
    s = lax.axis_index("s")
    wid = s * NC + c
    nstep = rpt // CHE
    pltpu.sync_copy(x0_hbm.at[pl.ds(wid * rpt, rpt)], i0_v)
    pltpu.sync_copy(x1_hbm.at[pl.ds(wid * rpt, rpt)], i1_v)

    def step(t, carry):
        sl = pl.ds(t * CHE, CHE)
        pltpu.async_copy(kt_hbm.at[i0_v.at[sl]], ka_v, sem0)
        pltpu.async_copy(vt_hbm.at[i1_v.at[sl]], vb_v, sem1)
        pltpu.make_async_copy(kt_hbm.at[i0_v.at[sl]], ka_v, sem0).wait()
        pltpu.make_async_copy(vt_hbm.at[i1_v.at[sl]], vb_v, sem1).wait()

        def addrow(r, carry2):
            for cc in range(H // L):
                sl = pl.ds(cc * L, L)
                ka_v[r, sl] = ka_v[r, sl] + vb_v[r, sl]
            return carry2

        lax.fori_loop(0, CHE, addrow, 0)
        pltpu.sync_copy(ka_v, out_hbm.at[pl.ds(wid * rpt + t * CHE, CHE)])
        return carry

    lax.fori_loop(0, nstep, step, 0)


def _seg_body(h_hbm, pk_hbm, out_hbm,
              pk_v, sidx_v, didx_v, rows_v, hh_sh, agg_sh, sem0, sem1,
              *, np_, wb):
    c = lax.axis_index("c")
    s = lax.axis_index("s")
    wid = s * NC + c
    half_n = np_ // 2
    agg_rows = half_n + ECH
    rps_h = half_n // NS
    nzch = agg_rows // ECH

    zero = jnp.zeros((L,), jnp.float32)

    def zrow(r, carry):
        for cc in range(H // L):
            rows_v[0, r, pl.ds(cc * L, L)] = zero
        return carry

    def zero_agg():
        lax.fori_loop(0, ECH, zrow, 0)

        def zcopy(k, carry):
            q = s + k * NS

            @pl.when(q < nzch)
            def _():
                pltpu.sync_copy(rows_v.at[0],
                                agg_sh.at[pl.ds(q * ECH, ECH)])
            return carry

        lax.fori_loop(0, -(-nzch // NS), zcopy, 0)

    def stage_h(sh):
        plsc.subcore_barrier()
        pltpu.sync_copy(h_hbm.at[pl.ds(sh * half_n + s * rps_h, rps_h)],
                        hh_sh.at[pl.ds(s * rps_h, rps_h)])
        plsc.subcore_barrier()

    def flush(dh):
        plsc.subcore_barrier()
        pltpu.sync_copy(agg_sh.at[pl.ds(s * rps_h, rps_h)],
                        out_hbm.at[c, pl.ds(dh * half_n + s * rps_h, rps_h)])

    def gstart(j, b):
        if b == 0:
            pltpu.async_copy(hh_sh.at[sidx_v.at[j]], rows_v.at[0], sem0)
        else:
            pltpu.async_copy(hh_sh.at[sidx_v.at[j]], rows_v.at[1], sem1)

    def gwait(j, b):
        if b == 0:
            pltpu.make_async_copy(hh_sh.at[sidx_v.at[j]], rows_v.at[0], sem0).wait()
        else:
            pltpu.make_async_copy(hh_sh.at[sidx_v.at[j]], rows_v.at[1], sem1).wait()

    def scat(j, b):
        pltpu.sync_copy(rows_v.at[b], agg_sh.at[didx_v.at[j]], add=True)

    def phase(b):
        pltpu.sync_copy(pk_hbm.at[pl.ds((b * NW + wid) * wb, wb)], pk_v)

        def unpack(j, carry):
            for cc in range(H // L):
                sl = pl.ds(cc * L, L)
                v = pk_v[j, sl]
                sidx_v[j, sl] = lax.shift_right_logical(v, 13)
                didx_v[j, sl] = lax.bitwise_and(v, 8191)
            return carry

        lax.fori_loop(0, wb, unpack, 0)

        gstart(0, 0)

        def pair(j2, carry):
            j = 2 * j2
            gstart(j + 1, 1)
            gwait(j, 0)
            scat(j, 0)

            @pl.when(j2 < wb // 2 - 1)
            def _():
                gstart(j + 2, 0)

            gwait(j + 1, 1)
            scat(j + 1, 1)
            return carry

        lax.fori_loop(0, wb // 2, pair, 0)

    zero_agg()
    stage_h(0)
    phase(0)
    stage_h(1)
    phase(2)
    flush(0)
    plsc.subcore_barrier()
    zero_agg()
    plsc.subcore_barrier()
    phase(3)
    stage_h(0)
    phase(1)
    flush(1)


def _make_emb(np_):
    rpt = np_ // NW
    mesh = plsc.VectorSubcoreMesh(core_axis_name="c", subcore_axis_name="s",
                                  num_cores=NC, num_subcores=NS)
    return pl.kernel(
        functools.partial(_emb_body, rpt=rpt),
        out_type=jax.ShapeDtypeStruct((np_, H), jnp.float32),
        mesh=mesh,
        scratch_types=[
            pltpu.VMEM((rpt,), jnp.int32),
            pltpu.VMEM((rpt,), jnp.int32),
            pltpu.VMEM((CHE, H), jnp.float32),
            pltpu.VMEM((CHE, H), jnp.float32),
            pltpu.SemaphoreType.DMA,
            pltpu.SemaphoreType.DMA,
        ],
    )


def _make_seg(np_, wb):
    mesh = plsc.VectorSubcoreMesh(core_axis_name="c", subcore_axis_name="s",
                                  num_cores=NC, num_subcores=NS)
    return pl.kernel(
        functools.partial(_seg_body, np_=np_, wb=wb),
        out_type=jax.ShapeDtypeStruct((NC, np_, H), jnp.float32),
        mesh=mesh,
        scratch_types=[
            pltpu.VMEM((wb, ECH), jnp.int32),
            pltpu.VMEM((wb, ECH), jnp.int32),
            pltpu.VMEM((wb, ECH), jnp.int32),
            pltpu.VMEM((2, ECH, H), jnp.float32),
            pltpu.VMEM_SHARED((np_ // 2, H), jnp.float32),
            pltpu.VMEM_SHARED((np_ // 2 + ECH, H), jnp.float32),
            pltpu.SemaphoreType.DMA,
            pltpu.SemaphoreType.DMA,
        ],
    )


def _dense_mid_body(h_ref, p0_ref, p1_ref, ws_ref, wn_ref, bgb_ref, out_ref):
    h = h_ref[...]
    agg = p0_ref[...] + p1_ref[...]
    t = (jnp.dot(h, ws_ref[...], preferred_element_type=jnp.float32)
         + jnp.dot(agg, wn_ref[...], preferred_element_type=jnp.float32)
         + bgb_ref[0:1, :])
    hn = h + jnp.maximum(t, 0.0)
    mu = jnp.mean(hn, axis=-1, keepdims=True)
    var = jnp.mean((hn - mu) ** 2, axis=-1, keepdims=True)
    out_ref[...] = ((hn - mu) / jnp.sqrt(var + 1e-5) * bgb_ref[1:2, :]
                    + bgb_ref[2:3, :])


def _dense_last_body(h_ref, p0_ref, p1_ref, ws_ref, wn_ref, bgb_ref,
                     wout_ref, mask_ref, out_ref):
    h = h_ref[...]
    agg = p0_ref[...] + p1_ref[...]
    t = (jnp.dot(h, ws_ref[...], preferred_element_type=jnp.float32)
         + jnp.dot(agg, wn_ref[...], preferred_element_type=jnp.float32)
         + bgb_ref[0:1, :])
    hn = h + jnp.maximum(t, 0.0)
    mu = jnp.mean(hn, axis=-1, keepdims=True)
    var = jnp.mean((hn - mu) ** 2, axis=-1, keepdims=True)
    hln = ((hn - mu) / jnp.sqrt(var + 1e-5) * bgb_ref[1:2, :]
           + bgb_ref[2:3, :])
    out_ref[...] = (jnp.dot(hln, wout_ref[...], preferred_element_type=jnp.float32)
                    * mask_ref[...])


def _dense_mid(h, p0, p1, ws, wn, bgb, *, np_, blk=2048):
    grid = (np_ // blk,)
    return pl.pallas_call(
        _dense_mid_body,
        grid=grid,
        in_specs=[
            pl.BlockSpec((blk, H), lambda i: (i, 0)),
            pl.BlockSpec((blk, H), lambda i: (i, 0)),
            pl.BlockSpec((blk, H), lambda i: (i, 0)),
            pl.BlockSpec((H, H), lambda i: (0, 0)),
            pl.BlockSpec((H, H), lambda i: (0, 0)),
            pl.BlockSpec((3, H), lambda i: (0, 0)),
        ],
        out_specs=pl.BlockSpec((blk, H), lambda i: (i, 0)),
        out_shape=jax.ShapeDtypeStruct((np_, H), jnp.float32),
    )(h, p0, p1, ws, wn, bgb)


def _dense_last(h, p0, p1, ws, wn, bgb, wout_t, maskf, *, np_, blk=2048):
    grid = (np_ // blk,)
    return pl.pallas_call(
        _dense_last_body,
        grid=grid,
        in_specs=[
            pl.BlockSpec((blk, H), lambda i: (i, 0)),
            pl.BlockSpec((blk, H), lambda i: (i, 0)),
            pl.BlockSpec((blk, H), lambda i: (i, 0)),
            pl.BlockSpec((H, H), lambda i: (0, 0)),
            pl.BlockSpec((H, H), lambda i: (0, 0)),
            pl.BlockSpec((3, H), lambda i: (0, 0)),
            pl.BlockSpec((H, H), lambda i: (0, 0)),
            pl.BlockSpec((blk, 1), lambda i: (i, 0)),
        ],
        out_specs=pl.BlockSpec((blk, H), lambda i: (i, 0)),
        out_shape=jax.ShapeDtypeStruct((np_, H), jnp.float32),
    )(h, p0, p1, ws, wn, bgb, wout_t, maskf)


def kernel(x, edge_index, batch, root_mask, key_table, val_table,
           Ws, Wn, bias, ln_g, ln_b, W_out):
    n, _ = x.shape
    e = edge_index.shape[1]
    depth = Ws.shape[0]

    np_ = ((n + NW * CHE - 1) // (NW * CHE)) * (NW * CHE)
    half = np_ // 2
    wb = -(-(3 * e) // (10 * NW * ECH))
    wb += wb % 2
    cap = wb * NW * ECH

    pad_n = np_ - n
    x0 = jnp.concatenate([x[:, 0], jnp.zeros((pad_n,), jnp.int32)])
    x1 = jnp.concatenate([x[:, 1], jnp.zeros((pad_n,), jnp.int32)])

    s0, d0 = edge_index[0], edge_index[1]
    sl = (s0 >= half).astype(jnp.int32)
    dl = (d0 >= half).astype(jnp.int32)
    key = sl * 2 + dl
    packed = (s0 - sl * half) * 8192 + (d0 - dl * half)
    ks, ps = jax.lax.sort_key_val(key, packed)
    starts = jnp.searchsorted(ks, jnp.arange(4, dtype=jnp.int32)).astype(jnp.int32)
    sizes = jnp.concatenate([starts[1:], jnp.array([e], jnp.int32)]) - starts
    j = jnp.arange(cap, dtype=jnp.int32)[None, :]
    idx = jnp.minimum(starts[:, None] + j, e - 1)
    valid = j < sizes[:, None]
    pk_def = ((j % jnp.int32(half)) * 8192
              + jnp.int32(half) + (j % jnp.int32(ECH)))
    pk = jnp.where(valid, ps[idx], pk_def)
    pk2d = pk.reshape(4 * cap // ECH, ECH)

    h = _make_emb(np_)(x0, x1, key_table, val_table)

    seg = _make_seg(np_, wb)
    maskf = jnp.concatenate([root_mask.astype(jnp.float32),
                             jnp.zeros((pad_n,), jnp.float32)]).reshape(np_, 1)
    wout_t = W_out.T

    preds = None
    for i in range(depth):
        parts = seg(h, pk2d)
        bgb = jnp.stack([bias[i], ln_g[i], ln_b[i]])
        if i < depth - 1:
            h = _dense_mid(h, parts[0], parts[1], Ws[i], Wn[i], bgb, np_=np_)
        else:
            preds = _dense_last(h, parts[0], parts[1], Ws[i], Wn[i], bgb,
                                wout_t, maskf, np_=np_)
    return preds[:n]


# speedup vs baseline: 2.7124x; 1.7135x over previous
import functools

import jax
import jax.numpy as jnp
from jax import lax
from jax.experimental import pallas as pl
from jax.experimental.pallas import tpu as pltpu
from jax.experimental.pallas import tpu_sc as plsc

NC = 2
NS = 16
NW = NC * NS
L = 16

H = 128
CHE = 64
ECH = 128


def _emb_body(x0_hbm, x1_hbm, kt_hbm, vt_hbm, out_hbm,
              i0_v, i1_v, ka_v, vb_v, sem0, sem1, *, rpt):
    c = lax.axis_index("c")
    s = lax.axis_index("s")
    wid = s * NC + c
    nstep = rpt // CHE
    pltpu.sync_copy(x0_hbm.at[pl.ds(wid * rpt, rpt)], i0_v)
    pltpu.sync_copy(x1_hbm.at[pl.ds(wid * rpt, rpt)], i1_v)

    def step(t, carry):
        sl = pl.ds(t * CHE, CHE)
        pltpu.async_copy(kt_hbm.at[i0_v.at[sl]], ka_v, sem0)
        pltpu.async_copy(vt_hbm.at[i1_v.at[sl]], vb_v, sem1)
        pltpu.make_async_copy(kt_hbm.at[i0_v.at[sl]], ka_v, sem0).wait()
        pltpu.make_async_copy(vt_hbm.at[i1_v.at[sl]], vb_v, sem1).wait()

        def addrow(r, carry2):
            for cc in range(H // L):
                sl = pl.ds(cc * L, L)
                ka_v[r, sl] = ka_v[r, sl] + vb_v[r, sl]
            return carry2

        lax.fori_loop(0, CHE, addrow, 0)
        pltpu.sync_copy(ka_v, out_hbm.at[pl.ds(wid * rpt + t * CHE, CHE)])
        return carry

    lax.fori_loop(0, nstep, step, 0)


def _part_body(s_hbm, d_hbm, pk_hbm, se_v, de_v, bb_v, *, np_, ept, capt):
    c = lax.axis_index("c")
    s = lax.axis_index("s")
    wid = s * NC + c
    half = np_ // 2

    pltpu.sync_copy(s_hbm.at[pl.ds(wid * ept, ept)], se_v)
    pltpu.sync_copy(d_hbm.at[pl.ds(wid * ept, ept)], de_v)

    iota = lax.iota(jnp.int32, L)

    def prefill(q, carry):
        jv = q * L + iota
        val = jv * 8192 + (jnp.int32(half) + lax.rem(jv, jnp.int32(ECH)))
        for b in range(4):
            bb_v[pl.ds(b * capt + q * L, L)] = val
        return carry

    lax.fori_loop(0, capt // L, prefill, 0)

    def chunk(i, offs):
        sv = se_v[pl.ds(i * L, L)]
        dv = de_v[pl.ds(i * L, L)]
        sli = lax.shift_right_arithmetic(sv - jnp.int32(half), 31) + 1
        dli = lax.shift_right_arithmetic(dv - jnp.int32(half), 31) + 1
        keyv = sli * 2 + dli
        pkv = (sv - sli * jnp.int32(half)) * 8192 + (dv - dli * jnp.int32(half))
        k0 = lax.bitwise_and(keyv, 1)
        k1 = lax.shift_right_logical(keyv, 1)
        pos = keyv * jnp.int32(capt)
        new_offs = []
        for b in range(4):
            eq = (k0 if b & 1 else 1 - k0) * (k1 if b & 2 else 1 - k1)
            rank = plsc.cumsum(eq) - 1
            pos = pos + eq * (jnp.minimum(offs[b], capt - L) + rank)
            new_offs.append(jnp.minimum(offs[b] + jnp.sum(eq), capt - L))
        plsc.store_scatter(bb_v, [pos], pkv)
        return tuple(new_offs)

    z = jnp.int32(0)
    lax.fori_loop(0, ept // L, chunk, (z, z, z, z))

    for b in range(4):
        pltpu.sync_copy(bb_v.at[pl.ds(b * capt, capt)],
                        pk_hbm.at[pl.ds((b * NW + wid) * capt, capt)])


def _make_part(np_, e, capt):
    ept = e // NW
    mesh = plsc.VectorSubcoreMesh(core_axis_name="c", subcore_axis_name="s",
                                  num_cores=NC, num_subcores=NS)
    return pl.kernel(
        functools.partial(_part_body, np_=np_, ept=ept, capt=capt),
        out_type=jax.ShapeDtypeStruct((4 * NW * capt,), jnp.int32),
        mesh=mesh,
        compiler_params=pltpu.CompilerParams(needs_layout_passes=False),
        scratch_types=[
            pltpu.VMEM((ept,), jnp.int32),
            pltpu.VMEM((ept,), jnp.int32),
            pltpu.VMEM((4 * capt,), jnp.int32),
        ],
    )


def _seg_body(h_hbm, pk_hbm, out_hbm,
              pk_v, sidx_v, didx_v, rows_v, hh_sh, agg_sh, sem0, sem1,
              *, np_, wb):
    c = lax.axis_index("c")
    s = lax.axis_index("s")
    wid = s * NC + c
    half_n = np_ // 2
    agg_rows = half_n + ECH
    rps_h = half_n // NS
    nzch = agg_rows // ECH

    zero = jnp.zeros((L,), jnp.float32)

    def zrow(r, carry):
        for cc in range(H // L):
            rows_v[0, r, pl.ds(cc * L, L)] = zero
        return carry

    def zero_agg():
        lax.fori_loop(0, ECH, zrow, 0)

        def zcopy(k, carry):
            q = s + k * NS

            @pl.when(q < nzch)
            def _():
                pltpu.sync_copy(rows_v.at[0],
                                agg_sh.at[pl.ds(q * ECH, ECH)])
            return carry

        lax.fori_loop(0, -(-nzch // NS), zcopy, 0)

    def stage_h(sh):
        plsc.subcore_barrier()
        pltpu.sync_copy(h_hbm.at[pl.ds(sh * half_n + s * rps_h, rps_h)],
                        hh_sh.at[pl.ds(s * rps_h, rps_h)])
        plsc.subcore_barrier()

    def flush(dh):
        plsc.subcore_barrier()
        pltpu.sync_copy(agg_sh.at[pl.ds(s * rps_h, rps_h)],
                        out_hbm.at[c, pl.ds(dh * half_n + s * rps_h, rps_h)])

    def gstart(j, b):
        if b == 0:
            pltpu.async_copy(hh_sh.at[sidx_v.at[j]], rows_v.at[0], sem0)
        else:
            pltpu.async_copy(hh_sh.at[sidx_v.at[j]], rows_v.at[1], sem1)

    def gwait(j, b):
        if b == 0:
            pltpu.make_async_copy(hh_sh.at[sidx_v.at[j]], rows_v.at[0], sem0).wait()
        else:
            pltpu.make_async_copy(hh_sh.at[sidx_v.at[j]], rows_v.at[1], sem1).wait()

    def scat(j, b):
        pltpu.sync_copy(rows_v.at[b], agg_sh.at[didx_v.at[j]], add=True)

    def phase(b):
        pltpu.sync_copy(pk_hbm.at[pl.ds((b * NW + wid) * wb * ECH, wb * ECH)],
                        pk_v)

        def unpack(j, carry):
            for cc in range(ECH // L):
                sl = pl.ds(cc * L, L)
                v = pk_v[pl.ds(j * ECH + cc * L, L)]
                sidx_v[j, sl] = lax.shift_right_logical(v, 13)
                didx_v[j, sl] = lax.bitwise_and(v, 8191)
            return carry

        lax.fori_loop(0, wb, unpack, 0)

        gstart(0, 0)

        def pair(j2, carry):
            j = 2 * j2
            gstart(j + 1, 1)
            gwait(j, 0)
            scat(j, 0)

            @pl.when(j2 < wb // 2 - 1)
            def _():
                gstart(j + 2, 0)

            gwait(j + 1, 1)
            scat(j + 1, 1)
            return carry

        lax.fori_loop(0, wb // 2, pair, 0)

    zero_agg()
    stage_h(0)
    phase(0)
    stage_h(1)
    phase(2)
    flush(0)
    plsc.subcore_barrier()
    zero_agg()
    plsc.subcore_barrier()
    phase(3)
    stage_h(0)
    phase(1)
    flush(1)


def _make_emb(np_):
    rpt = np_ // NW
    mesh = plsc.VectorSubcoreMesh(core_axis_name="c", subcore_axis_name="s",
                                  num_cores=NC, num_subcores=NS)
    return pl.kernel(
        functools.partial(_emb_body, rpt=rpt),
        out_type=jax.ShapeDtypeStruct((np_, H), jnp.float32),
        mesh=mesh,
        scratch_types=[
            pltpu.VMEM((rpt,), jnp.int32),
            pltpu.VMEM((rpt,), jnp.int32),
            pltpu.VMEM((CHE, H), jnp.float32),
            pltpu.VMEM((CHE, H), jnp.float32),
            pltpu.SemaphoreType.DMA,
            pltpu.SemaphoreType.DMA,
        ],
    )


def _make_seg(np_, wb):
    mesh = plsc.VectorSubcoreMesh(core_axis_name="c", subcore_axis_name="s",
                                  num_cores=NC, num_subcores=NS)
    return pl.kernel(
        functools.partial(_seg_body, np_=np_, wb=wb),
        out_type=jax.ShapeDtypeStruct((NC, np_, H), jnp.float32),
        mesh=mesh,
        scratch_types=[
            pltpu.VMEM((wb * ECH,), jnp.int32),
            pltpu.VMEM((wb, ECH), jnp.int32),
            pltpu.VMEM((wb, ECH), jnp.int32),
            pltpu.VMEM((2, ECH, H), jnp.float32),
            pltpu.VMEM_SHARED((np_ // 2, H), jnp.float32),
            pltpu.VMEM_SHARED((np_ // 2 + ECH, H), jnp.float32),
            pltpu.SemaphoreType.DMA,
            pltpu.SemaphoreType.DMA,
        ],
    )


def _dense_mid_body(h_ref, p0_ref, p1_ref, ws_ref, wn_ref, bgb_ref, out_ref):
    h = h_ref[...]
    agg = p0_ref[...] + p1_ref[...]
    t = (jnp.dot(h, ws_ref[...], preferred_element_type=jnp.float32)
         + jnp.dot(agg, wn_ref[...], preferred_element_type=jnp.float32)
         + bgb_ref[0:1, :])
    hn = h + jnp.maximum(t, 0.0)
    mu = jnp.mean(hn, axis=-1, keepdims=True)
    var = jnp.mean((hn - mu) ** 2, axis=-1, keepdims=True)
    out_ref[...] = ((hn - mu) / jnp.sqrt(var + 1e-5) * bgb_ref[1:2, :]
                    + bgb_ref[2:3, :])


def _dense_last_body(h_ref, p0_ref, p1_ref, ws_ref, wn_ref, bgb_ref,
                     wout_ref, mask_ref, out_ref):
    h = h_ref[...]
    agg = p0_ref[...] + p1_ref[...]
    t = (jnp.dot(h, ws_ref[...], preferred_element_type=jnp.float32)
         + jnp.dot(agg, wn_ref[...], preferred_element_type=jnp.float32)
         + bgb_ref[0:1, :])
    hn = h + jnp.maximum(t, 0.0)
    mu = jnp.mean(hn, axis=-1, keepdims=True)
    var = jnp.mean((hn - mu) ** 2, axis=-1, keepdims=True)
    hln = ((hn - mu) / jnp.sqrt(var + 1e-5) * bgb_ref[1:2, :]
           + bgb_ref[2:3, :])
    out_ref[...] = (jnp.dot(hln, wout_ref[...], preferred_element_type=jnp.float32)
                    * mask_ref[...])


def _dense_mid(h, p0, p1, ws, wn, bgb, *, np_, blk=2048):
    grid = (np_ // blk,)
    return pl.pallas_call(
        _dense_mid_body,
        grid=grid,
        in_specs=[
            pl.BlockSpec((blk, H), lambda i: (i, 0)),
            pl.BlockSpec((blk, H), lambda i: (i, 0)),
            pl.BlockSpec((blk, H), lambda i: (i, 0)),
            pl.BlockSpec((H, H), lambda i: (0, 0)),
            pl.BlockSpec((H, H), lambda i: (0, 0)),
            pl.BlockSpec((3, H), lambda i: (0, 0)),
        ],
        out_specs=pl.BlockSpec((blk, H), lambda i: (i, 0)),
        out_shape=jax.ShapeDtypeStruct((np_, H), jnp.float32),
    )(h, p0, p1, ws, wn, bgb)


def _dense_last(h, p0, p1, ws, wn, bgb, wout_t, maskf, *, np_, blk=2048):
    grid = (np_ // blk,)
    return pl.pallas_call(
        _dense_last_body,
        grid=grid,
        in_specs=[
            pl.BlockSpec((blk, H), lambda i: (i, 0)),
            pl.BlockSpec((blk, H), lambda i: (i, 0)),
            pl.BlockSpec((blk, H), lambda i: (i, 0)),
            pl.BlockSpec((H, H), lambda i: (0, 0)),
            pl.BlockSpec((H, H), lambda i: (0, 0)),
            pl.BlockSpec((3, H), lambda i: (0, 0)),
            pl.BlockSpec((H, H), lambda i: (0, 0)),
            pl.BlockSpec((blk, 1), lambda i: (i, 0)),
        ],
        out_specs=pl.BlockSpec((blk, H), lambda i: (i, 0)),
        out_shape=jax.ShapeDtypeStruct((np_, H), jnp.float32),
    )(h, p0, p1, ws, wn, bgb, wout_t, maskf)


def kernel(x, edge_index, batch, root_mask, key_table, val_table,
           Ws, Wn, bias, ln_g, ln_b, W_out):
    n, _ = x.shape
    e = edge_index.shape[1]
    depth = Ws.shape[0]

    np_ = ((n + NW * CHE - 1) // (NW * CHE)) * (NW * CHE)
    half = np_ // 2
    wb = -(-(3 * e) // (10 * NW * ECH))
    wb += wb % 2
    cap = wb * NW * ECH

    pad_n = np_ - n
    x0 = jnp.concatenate([x[:, 0], jnp.zeros((pad_n,), jnp.int32)])
    x1 = jnp.concatenate([x[:, 1], jnp.zeros((pad_n,), jnp.int32)])

    pe = -(-e // (NW * L)) * (NW * L)
    s0, d0 = edge_index[0], edge_index[1]
    if pe != e:
        s0 = jnp.concatenate(
            [s0, jnp.arange(pe - e, dtype=jnp.int32) % jnp.int32(n)])
        d0 = jnp.concatenate([d0, jnp.full((pe - e,), np_, jnp.int32)])
    capt = wb * ECH
    pk = _make_part(np_, pe, capt)(s0, d0)

    h = _make_emb(np_)(x0, x1, key_table, val_table)

    seg = _make_seg(np_, wb)
    maskf = jnp.concatenate([root_mask.astype(jnp.float32),
                             jnp.zeros((pad_n,), jnp.float32)]).reshape(np_, 1)
    wout_t = W_out.T

    preds = None
    for i in range(depth):
        parts = seg(h, pk)
        bgb = jnp.stack([bias[i], ln_g[i], ln_b[i]])
        if i < depth - 1:
            h = _dense_mid(h, parts[0], parts[1], Ws[i], Wn[i], bgb, np_=np_)
        else:
            preds = _dense_last(h, parts[0], parts[1], Ws[i], Wn[i], bgb,
                                wout_t, maskf, np_=np_)
    return preds[:n]
